# baseline (speedup 1.0000x reference)
import functools

import jax
import jax.numpy as jnp
from jax import lax
from jax.experimental import pallas as pl
from jax.experimental.pallas import tpu as pltpu
from jax.experimental.pallas import tpu_sc as plsc

NC = 2
NS = 16
L = 16
NW = NC * NS

B = 16384
D = 32
RPB = 8
PPR = D // 2
BPW = B // NW
CHUNK = 128
NCHUNK = BPW // CHUNK
GROUPS = CHUNK // L


def _transform(idx, ridx, cb):
    for k in range(NCHUNK):
        def body(g, carry, k=k):
            off = pl.multiple_of(g * L, L)
            v = idx.at[k][pl.ds(off, L)]
            ridx.at[k][pl.ds(off, L)] = lax.shift_right_logical(v, 3)
            cb.at[k][pl.ds(off, L)] = lax.shift_left(v & (RPB - 1), 4)
            return carry
        lax.fori_loop(0, GROUPS, body, 0)


def _chunk_dot(ubuf, ibuf, cbu, cbi, ub_v, ib_v, out_v, k):
    lanes = lax.iota(jnp.int32, L)
    obase = k * CHUNK

    def group(g, carry):
        off = pl.multiple_of(g * L, L)
        rows = off + lanes
        cu = cbu.at[k][pl.ds(off, L)]
        ci = cbi.at[k][pl.ds(off, L)]
        acc = (ub_v[pl.ds(obase + off, L)] + ib_v[pl.ds(obase + off, L)])
        for p in range(PPR):
            dd = (lanes + p) & (PPR - 1)
            pu = plsc.load_gather(ubuf, [rows, cu | dd])
            pi = plsc.load_gather(ibuf, [rows, ci | dd])
            ua, ub_half = plsc.unpack(plsc.bitcast(pu, jnp.bfloat16),
                                      format=plsc.PackFormat.INTERLEAVED)
            ia, ib_half = plsc.unpack(plsc.bitcast(pi, jnp.bfloat16),
                                      format=plsc.PackFormat.INTERLEAVED)
            acc = acc + ua * ia + ub_half * ib_half
        out_v[pl.ds(obase + off, L)] = acc
        return carry

    lax.fori_loop(0, GROUPS, group, 0)


def _mf_body(user_h, item_h, uneg_h, ineg_h, ul_h, il_h, ub_h, ib_h,
             pos_h, neg_h,
             u_idx, i_idx, un_idx, in_idx,
             u_rid, i_rid, un_rid, in_rid,
             u_cb, i_cb, un_cb, in_cb,
             ubuf, ibuf,
             ub_v, ib_v, unb_v, inb_v,
             pos_v, neg_v, sem_b, sem0, sem1):
    c = lax.axis_index("c")
    s = lax.axis_index("s")
    wid = s * NC + c
    base = pl.multiple_of(wid * BPW, BPW)

    for k in range(NCHUNK):
        off = pl.multiple_of(base + k * CHUNK, CHUNK)
        pltpu.sync_copy(user_h.at[pl.ds(off, CHUNK)], u_idx.at[k])
        pltpu.sync_copy(item_h.at[pl.ds(off, CHUNK)], i_idx.at[k])
        pltpu.sync_copy(uneg_h.at[pl.ds(off, CHUNK)], un_idx.at[k])
        pltpu.sync_copy(ineg_h.at[pl.ds(off, CHUNK)], in_idx.at[k])

    bias_cps = []
    for k in range(NCHUNK):
        r = pl.ds(k * CHUNK, CHUNK)
        bias_cps.append(pltpu.async_copy(ub_h.at[u_idx.at[k]], ub_v.at[r], sem_b))
        bias_cps.append(pltpu.async_copy(ib_h.at[i_idx.at[k]], ib_v.at[r], sem_b))
        bias_cps.append(pltpu.async_copy(ub_h.at[un_idx.at[k]], unb_v.at[r], sem_b))
        bias_cps.append(pltpu.async_copy(ib_h.at[in_idx.at[k]], inb_v.at[r], sem_b))

    _transform(u_idx, u_rid, u_cb)
    _transform(i_idx, i_rid, i_cb)
    _transform(un_idx, un_rid, un_cb)
    _transform(in_idx, in_rid, in_cb)

    sched = []
    for k in range(NCHUNK):
        sched.append((u_rid, i_rid, u_cb, i_cb, ub_v, ib_v, pos_v, k))
    for k in range(NCHUNK):
        sched.append((un_rid, in_rid, un_cb, in_cb, unb_v, inb_v, neg_v, k))
    sems = (sem0, sem1)

    def fire(t):
        rid_u, rid_i, _, _, _, _, _, k = sched[t]
        slot = t % 2
        sem = sems[slot]
        return (pltpu.async_copy(ul_h.at[rid_u.at[k]], ubuf.at[slot], sem),
                pltpu.async_copy(il_h.at[rid_i.at[k]], ibuf.at[slot], sem))

    inflight = fire(0)
    for cp in bias_cps:
        cp.wait()
    for t in range(len(sched)):
        nxt = fire(t + 1) if t + 1 < len(sched) else None
        for cp in inflight:
            cp.wait()
        _, _, cb_u, cb_i, ubias, ibias, out_v, k = sched[t]
        slot = t % 2
        _chunk_dot(ubuf.at[slot], ibuf.at[slot], cb_u, cb_i,
                   ubias, ibias, out_v, k)
        inflight = nxt

    pltpu.sync_copy(pos_v, pos_h.at[pl.ds(base, BPW)])
    pltpu.sync_copy(neg_v, neg_h.at[pl.ds(base, BPW)])


@functools.partial(
    pl.kernel,
    out_type=(jax.ShapeDtypeStruct((B,), jnp.float32),
              jax.ShapeDtypeStruct((B,), jnp.float32)),
    mesh=plsc.VectorSubcoreMesh(core_axis_name="c", subcore_axis_name="s"),
    scratch_types=[
        pltpu.VMEM((NCHUNK, CHUNK), jnp.int32),
        pltpu.VMEM((NCHUNK, CHUNK), jnp.int32),
        pltpu.VMEM((NCHUNK, CHUNK), jnp.int32),
        pltpu.VMEM((NCHUNK, CHUNK), jnp.int32),
        pltpu.VMEM((NCHUNK, CHUNK), jnp.int32),
        pltpu.VMEM((NCHUNK, CHUNK), jnp.int32),
        pltpu.VMEM((NCHUNK, CHUNK), jnp.int32),
        pltpu.VMEM((NCHUNK, CHUNK), jnp.int32),
        pltpu.VMEM((NCHUNK, CHUNK), jnp.int32),
        pltpu.VMEM((NCHUNK, CHUNK), jnp.int32),
        pltpu.VMEM((NCHUNK, CHUNK), jnp.int32),
        pltpu.VMEM((NCHUNK, CHUNK), jnp.int32),
        pltpu.VMEM((2, CHUNK, 128), jnp.int32),
        pltpu.VMEM((2, CHUNK, 128), jnp.int32),
        pltpu.VMEM((BPW,), jnp.float32),
        pltpu.VMEM((BPW,), jnp.float32),
        pltpu.VMEM((BPW,), jnp.float32),
        pltpu.VMEM((BPW,), jnp.float32),
        pltpu.VMEM((BPW,), jnp.float32),
        pltpu.VMEM((BPW,), jnp.float32),
        pltpu.SemaphoreType.DMA,
        pltpu.SemaphoreType.DMA,
        pltpu.SemaphoreType.DMA,
    ],
    compiler_params=pltpu.CompilerParams(needs_layout_passes=False,
                                         use_tc_tiling_on_sc=True),
)
def _mf_sc(*refs):
    _mf_body(*refs)


def kernel(user, item, user_neg, item_neg, user_latent, item_latent,
           user_biases, item_biases):
    ul = jax.lax.bitcast_convert_type(
        user_latent.astype(jnp.bfloat16).reshape(-1, D // 2, 2),
        jnp.int32).reshape(-1, 128)
    il = jax.lax.bitcast_convert_type(
        item_latent.astype(jnp.bfloat16).reshape(-1, D // 2, 2),
        jnp.int32).reshape(-1, 128)
    ub = user_biases.reshape(-1)
    ib = item_biases.reshape(-1)
    return _mf_sc(user.astype(jnp.int32), item.astype(jnp.int32),
                  user_neg.astype(jnp.int32), item_neg.astype(jnp.int32),
                  ul, il, ub, ib)


# speedup vs baseline: 2.2778x; 2.2778x over previous
import functools

import jax
import jax.numpy as jnp
from jax import lax
from jax.experimental import pallas as pl
from jax.experimental.pallas import tpu as pltpu
from jax.experimental.pallas import tpu_sc as plsc

NC = 2
NS = 16
L = 16
NW = NC * NS

B = 16384
D = 32
BPW = B // NW
CHUNK = 128
NCHUNK = BPW // CHUNK
GROUPS = BPW // L


def _dot_branch(u_rows, i_rows, ub_v, ib_v, out_v):
    lanes = lax.iota(jnp.int32, L)

    def group(g, carry):
        b0 = pl.multiple_of(g * L, L)
        rows = b0 + lanes
        acc = ub_v[pl.ds(b0, L)] + ib_v[pl.ds(b0, L)]
        for d in range(D):
            cols = (lanes + d) & (D - 1)
            acc = acc + (plsc.load_gather(u_rows, [rows, cols])
                         * plsc.load_gather(i_rows, [rows, cols]))
        out_v[pl.ds(b0, L)] = acc
        return carry

    lax.fori_loop(0, GROUPS, group, 0)


def _mf_body(user_h, item_h, uneg_h, ineg_h, ul_h, il_h, ub_h, ib_h,
             pos_h, neg_h,
             u_idx, i_idx, un_idx, in_idx,
             u_rows, i_rows, un_rows, in_rows,
             ub_v, ib_v, unb_v, inb_v,
             pos_v, neg_v, sem_pos, sem_neg):
    c = lax.axis_index("c")
    s = lax.axis_index("s")
    wid = s * NC + c
    base = pl.multiple_of(wid * BPW, BPW)

    for k in range(NCHUNK):
        off = pl.multiple_of(base + k * CHUNK, CHUNK)
        pltpu.sync_copy(user_h.at[pl.ds(off, CHUNK)], u_idx.at[k])
        pltpu.sync_copy(item_h.at[pl.ds(off, CHUNK)], i_idx.at[k])
        pltpu.sync_copy(uneg_h.at[pl.ds(off, CHUNK)], un_idx.at[k])
        pltpu.sync_copy(ineg_h.at[pl.ds(off, CHUNK)], in_idx.at[k])

    pos_cps = []
    neg_cps = []
    for k in range(NCHUNK):
        r = pl.ds(k * CHUNK, CHUNK)
        pos_cps.append(pltpu.async_copy(ul_h.at[u_idx.at[k]], u_rows.at[r], sem_pos))
        pos_cps.append(pltpu.async_copy(il_h.at[i_idx.at[k]], i_rows.at[r], sem_pos))
        pos_cps.append(pltpu.async_copy(ub_h.at[u_idx.at[k]], ub_v.at[r], sem_pos))
        pos_cps.append(pltpu.async_copy(ib_h.at[i_idx.at[k]], ib_v.at[r], sem_pos))
    for k in range(NCHUNK):
        r = pl.ds(k * CHUNK, CHUNK)
        neg_cps.append(pltpu.async_copy(ul_h.at[un_idx.at[k]], un_rows.at[r], sem_neg))
        neg_cps.append(pltpu.async_copy(il_h.at[in_idx.at[k]], in_rows.at[r], sem_neg))
        neg_cps.append(pltpu.async_copy(ub_h.at[un_idx.at[k]], unb_v.at[r], sem_neg))
        neg_cps.append(pltpu.async_copy(ib_h.at[in_idx.at[k]], inb_v.at[r], sem_neg))

    for cp in pos_cps:
        cp.wait()
    _dot_branch(u_rows, i_rows, ub_v, ib_v, pos_v)
    for cp in neg_cps:
        cp.wait()
    _dot_branch(un_rows, in_rows, unb_v, inb_v, neg_v)

    pltpu.sync_copy(pos_v, pos_h.at[pl.ds(base, BPW)])
    pltpu.sync_copy(neg_v, neg_h.at[pl.ds(base, BPW)])


@functools.partial(
    pl.kernel,
    out_type=(jax.ShapeDtypeStruct((B,), jnp.float32),
              jax.ShapeDtypeStruct((B,), jnp.float32)),
    mesh=plsc.VectorSubcoreMesh(core_axis_name="c", subcore_axis_name="s"),
    scratch_types=[
        pltpu.VMEM((NCHUNK, CHUNK), jnp.int32),
        pltpu.VMEM((NCHUNK, CHUNK), jnp.int32),
        pltpu.VMEM((NCHUNK, CHUNK), jnp.int32),
        pltpu.VMEM((NCHUNK, CHUNK), jnp.int32),
        pltpu.VMEM((BPW, D), jnp.float32),
        pltpu.VMEM((BPW, D), jnp.float32),
        pltpu.VMEM((BPW, D), jnp.float32),
        pltpu.VMEM((BPW, D), jnp.float32),
        pltpu.VMEM((BPW,), jnp.float32),
        pltpu.VMEM((BPW,), jnp.float32),
        pltpu.VMEM((BPW,), jnp.float32),
        pltpu.VMEM((BPW,), jnp.float32),
        pltpu.VMEM((BPW,), jnp.float32),
        pltpu.VMEM((BPW,), jnp.float32),
        pltpu.SemaphoreType.DMA,
        pltpu.SemaphoreType.DMA,
    ],
    compiler_params=pltpu.CompilerParams(needs_layout_passes=False,
                                         use_tc_tiling_on_sc=False),
)
def _mf_sc(*refs):
    _mf_body(*refs)


def kernel(user, item, user_neg, item_neg, user_latent, item_latent,
           user_biases, item_biases):
    ub = user_biases.reshape(-1)
    ib = item_biases.reshape(-1)
    return _mf_sc(user.astype(jnp.int32), item.astype(jnp.int32),
                  user_neg.astype(jnp.int32), item_neg.astype(jnp.int32),
                  user_latent, item_latent, ub, ib)
